# baseline (speedup 1.0000x reference)
import jax
import jax.numpy as jnp
from jax.experimental import pallas as pl
from jax.experimental.pallas import tpu as pltpu

_LANES = 128
_SIZES = (8192, 24576, 32768, 32768, 24576, 8192)
_SLOT = (0, 1, 2, 0, 1, 2)
_SLOT_SIZES = (32768, 32768, 32768)
_W = 3
_OFFS = tuple(sum(_SIZES[:i]) for i in range(len(_SIZES)))


def _body(vecs, ids, ones, queue, qids, valid8, oq, oids, oval, *scr):
    batch = vecs.shape[0]
    nc = len(_SIZES)
    ns = len(_SLOT_SIZES)
    qb, ib, vb = scr[0:ns], scr[ns:2 * ns], scr[2 * ns:3 * ns]
    sin, sout = scr[3 * ns], scr[3 * ns + 1]

    def in_copies(c):
        lo, hi = _OFFS[c], _OFFS[c] + _SIZES[c]
        cps, slot = [], 0
        for seg_lo, seg_hi, q_src, i_src, v_src in (
                (lo, min(hi, batch), vecs, ids, ones),
                (max(lo, batch), hi, queue, qids, valid8)):
            if seg_lo >= seg_hi:
                continue
            n = seg_hi - seg_lo
            n2 = n // _LANES
            d0 = seg_lo - lo
            d2 = d0 // _LANES
            s2 = seg_lo // _LANES
            sl = _SLOT[c]
            cps += [
                pltpu.make_async_copy(q_src.at[pl.ds(seg_lo, n)],
                                      qb[sl].at[pl.ds(d0, n)], sin.at[slot + 0, c]),
                pltpu.make_async_copy(i_src.at[pl.ds(s2, n2)],
                                      ib[sl].at[pl.ds(d2, n2)], sin.at[slot + 1, c]),
                pltpu.make_async_copy(v_src.at[pl.ds(s2, n2)],
                                      vb[sl].at[pl.ds(d2, n2)], sin.at[slot + 2, c]),
            ]
            slot += 3
        return cps

    def out_copies(c):
        off, off2 = _OFFS[c], _OFFS[c] // _LANES
        n, n2 = _SIZES[c], _SIZES[c] // _LANES
        sl = _SLOT[c]
        return (
            pltpu.make_async_copy(qb[sl].at[pl.ds(0, n)],
                                  oq.at[pl.ds(off, n)], sout.at[0, c]),
            pltpu.make_async_copy(ib[sl].at[pl.ds(0, n2)],
                                  oids.at[pl.ds(off2, n2)], sout.at[1, c]),
            pltpu.make_async_copy(vb[sl].at[pl.ds(0, n2)],
                                  oval.at[pl.ds(off2, n2)], sout.at[2, c]),
        )

    reuse, seen = {}, {}
    for c, s in enumerate(_SLOT):
        if s in seen:
            reuse[c] = seen[s]
        seen[s] = c

    for c in range(min(_W, nc)):
        for cp in in_copies(c):
            cp.start()
    for c in range(nc):
        for cp in in_copies(c):
            cp.wait()
        for cp in out_copies(c):
            cp.start()
        nxt = c + _W
        if nxt < nc:
            if nxt in reuse:
                for cp in out_copies(reuse[nxt]):
                    cp.wait()
            for cp in in_copies(nxt):
                cp.start()
    waited = {reuse[nxt] for nxt in reuse if nxt < nc}
    for c in range(nc):
        if c not in waited:
            for cp in out_copies(c):
                cp.wait()


def kernel(vecs, ids, queue, queue_ids, valid):
    batch, dim = vecs.shape
    k = queue.shape[0]
    nc = len(_SIZES)
    ids2d = ids.reshape(batch // _LANES, _LANES)
    ones2d = jnp.ones((batch // _LANES, _LANES), dtype=jnp.int8)
    qids2d = queue_ids.reshape(k // _LANES, _LANES)
    valid8 = valid.astype(jnp.int8).reshape(k // _LANES, _LANES)

    hbm = pl.BlockSpec(memory_space=pltpu.MemorySpace.HBM)
    scratch = (
        [pltpu.VMEM((sz, dim), queue.dtype) for sz in _SLOT_SIZES]
        + [pltpu.VMEM((sz // _LANES, _LANES), queue_ids.dtype) for sz in _SLOT_SIZES]
        + [pltpu.VMEM((sz // _LANES, _LANES), jnp.int8) for sz in _SLOT_SIZES]
        + [pltpu.SemaphoreType.DMA((6, nc)), pltpu.SemaphoreType.DMA((3, nc))]
    )
    oq, oids2d, oval8 = pl.pallas_call(
        _body,
        in_specs=[hbm] * 6,
        out_specs=[hbm] * 3,
        out_shape=[
            jax.ShapeDtypeStruct((k, dim), queue.dtype),
            jax.ShapeDtypeStruct((k // _LANES, _LANES), queue_ids.dtype),
            jax.ShapeDtypeStruct((k // _LANES, _LANES), jnp.int8),
        ],
        scratch_shapes=scratch,
    )(vecs, ids2d, ones2d, queue, qids2d, valid8)
    return (oq, oids2d.reshape(k), oval8.reshape(k).astype(jnp.bool_))


# speedup vs baseline: 1.0034x; 1.0034x over previous
import jax
import jax.numpy as jnp
from jax.experimental import pallas as pl
from jax.experimental.pallas import tpu as pltpu

_LANES = 128
_SIZES = (16384, 32768, 32768, 32768, 16384)
_SLOT = (0, 1, 2, 0, 1)
_SLOT_SIZES = (32768, 32768, 32768)
_W = 2
_OFFS = tuple(sum(_SIZES[:i]) for i in range(len(_SIZES)))


def _body(vecs, ids, ones, queue, qids, valid8, oq, oids, oval, *scr):
    batch = vecs.shape[0]
    nc = len(_SIZES)
    ns = len(_SLOT_SIZES)
    qb, ib, vb = scr[0:ns], scr[ns:2 * ns], scr[2 * ns:3 * ns]
    sin, sout = scr[3 * ns], scr[3 * ns + 1]

    def in_copies(c):
        lo, hi = _OFFS[c], _OFFS[c] + _SIZES[c]
        cps, slot = [], 0
        for seg_lo, seg_hi, q_src, i_src, v_src in (
                (lo, min(hi, batch), vecs, ids, ones),
                (max(lo, batch), hi, queue, qids, valid8)):
            if seg_lo >= seg_hi:
                continue
            n = seg_hi - seg_lo
            n2 = n // _LANES
            d0 = seg_lo - lo
            d2 = d0 // _LANES
            s2 = seg_lo // _LANES
            sl = _SLOT[c]
            cps += [
                pltpu.make_async_copy(q_src.at[pl.ds(seg_lo, n)],
                                      qb[sl].at[pl.ds(d0, n)], sin.at[slot + 0, c]),
                pltpu.make_async_copy(i_src.at[pl.ds(s2, n2)],
                                      ib[sl].at[pl.ds(d2, n2)], sin.at[slot + 1, c]),
                pltpu.make_async_copy(v_src.at[pl.ds(s2, n2)],
                                      vb[sl].at[pl.ds(d2, n2)], sin.at[slot + 2, c]),
            ]
            slot += 3
        return cps

    def out_copies(c):
        off, off2 = _OFFS[c], _OFFS[c] // _LANES
        n, n2 = _SIZES[c], _SIZES[c] // _LANES
        sl = _SLOT[c]
        return (
            pltpu.make_async_copy(qb[sl].at[pl.ds(0, n)],
                                  oq.at[pl.ds(off, n)], sout.at[0, c]),
            pltpu.make_async_copy(ib[sl].at[pl.ds(0, n2)],
                                  oids.at[pl.ds(off2, n2)], sout.at[1, c]),
            pltpu.make_async_copy(vb[sl].at[pl.ds(0, n2)],
                                  oval.at[pl.ds(off2, n2)], sout.at[2, c]),
        )

    reuse, seen = {}, {}
    for c, s in enumerate(_SLOT):
        if s in seen:
            reuse[c] = seen[s]
        seen[s] = c

    for c in range(min(_W, nc)):
        for cp in in_copies(c):
            cp.start()
    for c in range(nc):
        for cp in in_copies(c):
            cp.wait()
        for cp in out_copies(c):
            cp.start()
        nxt = c + _W
        if nxt < nc:
            if nxt in reuse:
                for cp in out_copies(reuse[nxt]):
                    cp.wait()
            for cp in in_copies(nxt):
                cp.start()
    waited = {reuse[nxt] for nxt in reuse if nxt < nc}
    for c in range(nc):
        if c not in waited:
            for cp in out_copies(c):
                cp.wait()


def kernel(vecs, ids, queue, queue_ids, valid):
    batch, dim = vecs.shape
    k = queue.shape[0]
    nc = len(_SIZES)
    ids2d = ids.reshape(batch // _LANES, _LANES)
    ones2d = jnp.ones((batch // _LANES, _LANES), dtype=jnp.int8)
    qids2d = queue_ids.reshape(k // _LANES, _LANES)
    valid8 = valid.astype(jnp.int8).reshape(k // _LANES, _LANES)

    hbm = pl.BlockSpec(memory_space=pltpu.MemorySpace.HBM)
    scratch = (
        [pltpu.VMEM((sz, dim), queue.dtype) for sz in _SLOT_SIZES]
        + [pltpu.VMEM((sz // _LANES, _LANES), queue_ids.dtype) for sz in _SLOT_SIZES]
        + [pltpu.VMEM((sz // _LANES, _LANES), jnp.int8) for sz in _SLOT_SIZES]
        + [pltpu.SemaphoreType.DMA((6, nc)), pltpu.SemaphoreType.DMA((3, nc))]
    )
    oq, oids2d, oval8 = pl.pallas_call(
        _body,
        in_specs=[hbm] * 6,
        out_specs=[hbm] * 3,
        out_shape=[
            jax.ShapeDtypeStruct((k, dim), queue.dtype),
            jax.ShapeDtypeStruct((k // _LANES, _LANES), queue_ids.dtype),
            jax.ShapeDtypeStruct((k // _LANES, _LANES), jnp.int8),
        ],
        scratch_shapes=scratch,
    )(vecs, ids2d, ones2d, queue, qids2d, valid8)
    return (oq, oids2d.reshape(k), oval8.reshape(k).astype(jnp.bool_))
